# baseline (device time: 26275 ns/iter reference)
import jax
import jax.numpy as jnp
from jax import lax
from jax.experimental import pallas as pl
from jax.experimental.pallas import tpu as pltpu

N_DEV = 4
T = 512
D = 512
F = 1024
E_LOC = 2
C = 160


def kernel(x, assign, W1, W2):
    a_col = assign.reshape(T, 1)

    def body(x_ref, a_ref, w1_ref, w2_ref, out_ref,
             xpk, apk, xrx, arx, yloc, yrx, w1f, w2f, w1b, w2b,
             x_send, x_recv, a_send, a_recv, y_send, y_recv,
             w1_sem, w2_sem):
        my = lax.axis_index("i")

        w1cp = pltpu.make_async_copy(w1_ref, w1f, w1_sem)
        w2cp = pltpu.make_async_copy(w2_ref, w2f, w2_sem)
        w1cp.start()
        w2cp.start()

        barrier = pltpu.get_barrier_semaphore()
        for r in range(1, N_DEV):
            tgt = lax.rem(my + r, N_DEV)
            pl.semaphore_signal(barrier, inc=1, device_id=(tgt,),
                                device_id_type=pl.DeviceIdType.MESH)
        pl.semaphore_wait(barrier, N_DEV - 1)

        a_c = a_ref[...]
        dest_c = a_c // E_LOC
        onehot = (dest_c == lax.broadcasted_iota(jnp.int32, (T, N_DEV), 1))
        tri = (lax.broadcasted_iota(jnp.int32, (T, T), 0)
               > lax.broadcasted_iota(jnp.int32, (T, T), 1))
        rank = jnp.dot(tri.astype(jnp.bfloat16), onehot.astype(jnp.bfloat16),
                       preferred_element_type=jnp.float32)
        myrank = jnp.sum(jnp.where(onehot, rank, 0.0), axis=1,
                         keepdims=True).astype(jnp.int32)
        slot_col = dest_c * C + myrank
        slot_row = slot_col.reshape(1, T)

        rows = lax.broadcasted_iota(jnp.int32, (N_DEV * C, T), 0)
        P = (rows == slot_row).astype(jnp.bfloat16)
        xb = x_ref[...].astype(jnp.bfloat16)
        xpk[...] = jnp.dot(P, xb,
                           preferred_element_type=jnp.float32
                           ).astype(jnp.bfloat16)
        a1b = (a_c + 1).astype(jnp.bfloat16)
        apk[...] = jnp.dot(P, a1b,
                           preferred_element_type=jnp.float32
                           ).astype(jnp.bfloat16)

        drain = []
        for r in range(1, N_DEV):
            tgt = lax.rem(my + r, N_DEV)
            xs = pltpu.make_async_remote_copy(
                src_ref=xpk.at[pl.ds(tgt * C, C), :],
                dst_ref=xrx.at[pl.ds(my * C, C), :],
                send_sem=x_send.at[r - 1],
                recv_sem=x_recv.at[r - 1],
                device_id=(tgt,),
                device_id_type=pl.DeviceIdType.MESH,
            )
            sa = pltpu.make_async_remote_copy(
                src_ref=apk.at[pl.ds(tgt * C, C), :],
                dst_ref=arx.at[pl.ds(my * C, C), :],
                send_sem=a_send.at[r - 1],
                recv_sem=a_recv.at[r - 1],
                device_id=(tgt,),
                device_id_type=pl.DeviceIdType.MESH,
            )
            xs.start()
            sa.start()
            drain += [xs, sa]

        w1cp.wait()
        w2cp.wait()
        w1b[...] = w1f[...].astype(jnp.bfloat16)
        w2b[...] = w2f[...].astype(jnp.bfloat16)

        def compute_chunk(xa, aa):
            acc = jnp.zeros((C, D), jnp.float32)
            for k in range(E_LOC):
                eid1 = (my * E_LOC + k + 1).astype(jnp.bfloat16)
                xm = jnp.where(aa == eid1, xa,
                               jnp.bfloat16(0.0)).astype(jnp.bfloat16)
                h = jnp.maximum(
                    jnp.dot(xm, w1b[k],
                            preferred_element_type=jnp.float32),
                    0.0).astype(jnp.bfloat16)
                acc = acc + jnp.dot(h, w2b[k],
                                    preferred_element_type=jnp.float32)
            return acc.astype(jnp.bfloat16)

        yrx[pl.ds(my * C, C), :] = compute_chunk(
            xpk[pl.ds(my * C, C), :], apk[pl.ds(my * C, C), :])

        for r in range(1, N_DEV):
            src_pos = lax.rem(my - r + N_DEV, N_DEV)
            pltpu.make_async_remote_copy(
                src_ref=xpk.at[pl.ds(0, C), :],
                dst_ref=xrx.at[pl.ds(src_pos * C, C), :],
                send_sem=x_send.at[r - 1], recv_sem=x_recv.at[r - 1],
                device_id=(0,), device_id_type=pl.DeviceIdType.MESH,
            ).wait_recv()
            pltpu.make_async_remote_copy(
                src_ref=apk.at[pl.ds(0, C), :],
                dst_ref=arx.at[pl.ds(src_pos * C, C), :],
                send_sem=a_send.at[r - 1], recv_sem=a_recv.at[r - 1],
                device_id=(0,), device_id_type=pl.DeviceIdType.MESH,
            ).wait_recv()

            yloc[pl.ds(src_pos * C, C), :] = compute_chunk(
                xrx[pl.ds(src_pos * C, C), :],
                arx[pl.ds(src_pos * C, C), :])
            yd = pltpu.make_async_remote_copy(
                src_ref=yloc.at[pl.ds(src_pos * C, C), :],
                dst_ref=yrx.at[pl.ds(my * C, C), :],
                send_sem=y_send.at[r - 1],
                recv_sem=y_recv.at[(N_DEV - r) - 1],
                device_id=(src_pos,),
                device_id_type=pl.DeviceIdType.MESH,
            )
            yd.start()
            drain.append(yd)

        for rc in range(1, N_DEV):
            src_pos = lax.rem(my - rc + N_DEV, N_DEV)
            pltpu.make_async_remote_copy(
                src_ref=yloc.at[pl.ds(0, C), :],
                dst_ref=yrx.at[pl.ds(src_pos * C, C), :],
                send_sem=y_send.at[rc - 1],
                recv_sem=y_recv.at[rc - 1],
                device_id=(0,),
                device_id_type=pl.DeviceIdType.MESH,
            ).wait_recv()

        out_ref[...] = lax.dot_general(
            P, yrx[...], dimension_numbers=(((0,), (0,)), ((), ())),
            preferred_element_type=jnp.float32)

        for d in drain:
            d.wait_send()

    return pl.pallas_call(
        body,
        out_shape=jax.ShapeDtypeStruct((T, D), jnp.float32),
        in_specs=[pl.BlockSpec(memory_space=pltpu.VMEM),
                  pl.BlockSpec(memory_space=pltpu.VMEM),
                  pl.BlockSpec(memory_space=pltpu.MemorySpace.HBM),
                  pl.BlockSpec(memory_space=pltpu.MemorySpace.HBM)],
        out_specs=pl.BlockSpec(memory_space=pltpu.VMEM),
        scratch_shapes=[
            pltpu.VMEM((N_DEV * C, D), jnp.bfloat16),
            pltpu.VMEM((N_DEV * C, 1), jnp.bfloat16),
            pltpu.VMEM((N_DEV * C, D), jnp.bfloat16),
            pltpu.VMEM((N_DEV * C, 1), jnp.bfloat16),
            pltpu.VMEM((N_DEV * C, D), jnp.bfloat16),
            pltpu.VMEM((N_DEV * C, D), jnp.bfloat16),
            pltpu.VMEM((E_LOC, D, F), jnp.float32),
            pltpu.VMEM((E_LOC, F, D), jnp.float32),
            pltpu.VMEM((E_LOC, D, F), jnp.bfloat16),
            pltpu.VMEM((E_LOC, F, D), jnp.bfloat16),
            pltpu.SemaphoreType.DMA((N_DEV - 1,)),
            pltpu.SemaphoreType.DMA((N_DEV - 1,)),
            pltpu.SemaphoreType.DMA((N_DEV - 1,)),
            pltpu.SemaphoreType.DMA((N_DEV - 1,)),
            pltpu.SemaphoreType.DMA((N_DEV - 1,)),
            pltpu.SemaphoreType.DMA((N_DEV - 1,)),
            pltpu.SemaphoreType.DMA,
            pltpu.SemaphoreType.DMA,
        ],
        compiler_params=pltpu.CompilerParams(collective_id=0),
    )(x, a_col, W1, W2)


# device time: 24852 ns/iter; 1.0573x vs baseline; 1.0573x over previous
import jax
import jax.numpy as jnp
from jax import lax
from jax.experimental import pallas as pl
from jax.experimental.pallas import tpu as pltpu

N_DEV = 4
T = 512
D = 512
F = 1024
E_LOC = 2
N_EXP = N_DEV * E_LOC
CE = 80
B = E_LOC * CE


def kernel(x, assign, W1, W2):
    a_col = assign.reshape(T, 1)

    def body(x_ref, a_ref, w1_ref, w2_ref, out_ref,
             xpk, xrx, yloc, yrx, w1f, w2f, w1b, w2b,
             x_send, x_recv, y_send, y_recv, w1_sem, w2_sem):
        my = lax.axis_index("i")

        w1cp = pltpu.make_async_copy(w1_ref, w1f, w1_sem)
        w2cp = pltpu.make_async_copy(w2_ref, w2f, w2_sem)
        w1cp.start()
        w2cp.start()

        barrier = pltpu.get_barrier_semaphore()
        for r in range(1, N_DEV):
            tgt = lax.rem(my + r, N_DEV)
            pl.semaphore_signal(barrier, inc=1, device_id=(tgt,),
                                device_id_type=pl.DeviceIdType.MESH)
        pl.semaphore_wait(barrier, N_DEV - 1)

        a_c = a_ref[...]
        onehot = (a_c == lax.broadcasted_iota(jnp.int32, (T, N_EXP), 1))
        tri = (lax.broadcasted_iota(jnp.int32, (T, T), 0)
               > lax.broadcasted_iota(jnp.int32, (T, T), 1))
        rank = jnp.dot(tri.astype(jnp.bfloat16), onehot.astype(jnp.bfloat16),
                       preferred_element_type=jnp.float32)
        myrank = jnp.sum(jnp.where(onehot, rank, 0.0), axis=1,
                         keepdims=True).astype(jnp.int32)
        slot_col = a_c * CE + myrank
        slot_row = slot_col.reshape(1, T)

        rows = lax.broadcasted_iota(jnp.int32, (N_EXP * CE, T), 0)
        P = (rows == slot_row).astype(jnp.bfloat16)
        xb = x_ref[...].astype(jnp.bfloat16)
        xpk[...] = jnp.dot(P, xb,
                           preferred_element_type=jnp.float32
                           ).astype(jnp.bfloat16)

        drain = []
        for r in range(1, N_DEV):
            tgt = lax.rem(my + r, N_DEV)
            xs = pltpu.make_async_remote_copy(
                src_ref=xpk.at[pl.ds(tgt * B, B), :],
                dst_ref=xrx.at[pl.ds(my * B, B), :],
                send_sem=x_send.at[r - 1],
                recv_sem=x_recv.at[r - 1],
                device_id=(tgt,),
                device_id_type=pl.DeviceIdType.MESH,
            )
            xs.start()
            drain.append(xs)

        w1cp.wait()
        w2cp.wait()
        w1b[...] = w1f[...].astype(jnp.bfloat16)
        w2b[...] = w2f[...].astype(jnp.bfloat16)

        def compute_block(in_ref, out_ref_, base):
            for k in range(E_LOC):
                xa = in_ref[pl.ds(base + k * CE, CE), :]
                h = jnp.maximum(
                    jnp.dot(xa, w1b[k],
                            preferred_element_type=jnp.float32),
                    0.0).astype(jnp.bfloat16)
                y = jnp.dot(h, w2b[k], preferred_element_type=jnp.float32)
                out_ref_[pl.ds(base + k * CE, CE), :] = (
                    y.astype(jnp.bfloat16))

        compute_block(xpk, yrx, my * B)

        for r in range(1, N_DEV):
            src_pos = lax.rem(my - r + N_DEV, N_DEV)
            pltpu.make_async_remote_copy(
                src_ref=xpk.at[pl.ds(0, B), :],
                dst_ref=xrx.at[pl.ds(src_pos * B, B), :],
                send_sem=x_send.at[r - 1], recv_sem=x_recv.at[r - 1],
                device_id=(0,), device_id_type=pl.DeviceIdType.MESH,
            ).wait_recv()

            compute_block(xrx, yloc, src_pos * B)
            yd = pltpu.make_async_remote_copy(
                src_ref=yloc.at[pl.ds(src_pos * B, B), :],
                dst_ref=yrx.at[pl.ds(my * B, B), :],
                send_sem=y_send.at[r - 1],
                recv_sem=y_recv.at[(N_DEV - r) - 1],
                device_id=(src_pos,),
                device_id_type=pl.DeviceIdType.MESH,
            )
            yd.start()
            drain.append(yd)

        for rc in range(1, N_DEV):
            src_pos = lax.rem(my - rc + N_DEV, N_DEV)
            pltpu.make_async_remote_copy(
                src_ref=yloc.at[pl.ds(0, B), :],
                dst_ref=yrx.at[pl.ds(src_pos * B, B), :],
                send_sem=y_send.at[rc - 1],
                recv_sem=y_recv.at[rc - 1],
                device_id=(0,),
                device_id_type=pl.DeviceIdType.MESH,
            ).wait_recv()

        out_ref[...] = lax.dot_general(
            P, yrx[...], dimension_numbers=(((0,), (0,)), ((), ())),
            preferred_element_type=jnp.float32)

        for d in drain:
            d.wait_send()

    return pl.pallas_call(
        body,
        out_shape=jax.ShapeDtypeStruct((T, D), jnp.float32),
        in_specs=[pl.BlockSpec(memory_space=pltpu.VMEM),
                  pl.BlockSpec(memory_space=pltpu.VMEM),
                  pl.BlockSpec(memory_space=pltpu.MemorySpace.HBM),
                  pl.BlockSpec(memory_space=pltpu.MemorySpace.HBM)],
        out_specs=pl.BlockSpec(memory_space=pltpu.VMEM),
        scratch_shapes=[
            pltpu.VMEM((N_EXP * CE, D), jnp.bfloat16),
            pltpu.VMEM((N_EXP * CE, D), jnp.bfloat16),
            pltpu.VMEM((N_EXP * CE, D), jnp.bfloat16),
            pltpu.VMEM((N_EXP * CE, D), jnp.bfloat16),
            pltpu.VMEM((E_LOC, D, F), jnp.float32),
            pltpu.VMEM((E_LOC, F, D), jnp.float32),
            pltpu.VMEM((E_LOC, D, F), jnp.bfloat16),
            pltpu.VMEM((E_LOC, F, D), jnp.bfloat16),
            pltpu.SemaphoreType.DMA((N_DEV - 1,)),
            pltpu.SemaphoreType.DMA((N_DEV - 1,)),
            pltpu.SemaphoreType.DMA((N_DEV - 1,)),
            pltpu.SemaphoreType.DMA((N_DEV - 1,)),
            pltpu.SemaphoreType.DMA,
            pltpu.SemaphoreType.DMA,
        ],
        compiler_params=pltpu.CompilerParams(collective_id=0),
    )(x, a_col, W1, W2)


# device time: 24231 ns/iter; 1.0844x vs baseline; 1.0256x over previous
import jax
import jax.numpy as jnp
from jax import lax
from jax.experimental import pallas as pl
from jax.experimental.pallas import tpu as pltpu

N_DEV = 4
T = 512
D = 512
F = 1024
E_LOC = 2
N_EXP = N_DEV * E_LOC
CE = 80
B = E_LOC * CE


def kernel(x, assign, W1, W2):
    a_row = assign.reshape(1, T)

    def body(x_ref, a_ref, w1_ref, w2_ref, out_ref,
             xpk, xrx, yloc, yrx, p_ref, w1f, w2f, w1b, w2b,
             x_send, x_recv, y_send, y_recv, w1_sem, w2_sem):
        my = lax.axis_index("i")

        w1cp = pltpu.make_async_copy(w1_ref, w1f, w1_sem)
        w2cp = pltpu.make_async_copy(w2_ref, w2f, w2_sem)
        w1cp.start()
        w2cp.start()

        barrier = pltpu.get_barrier_semaphore()
        for r in range(1, N_DEV):
            tgt = lax.rem(my + r, N_DEV)
            pl.semaphore_signal(barrier, inc=1, device_id=(tgt,),
                                device_id_type=pl.DeviceIdType.MESH)
        pl.semaphore_wait(barrier, N_DEV - 1)

        a_r = a_ref[...]
        onehot = (a_r == lax.broadcasted_iota(jnp.int32, (N_EXP, T), 0))
        triu = (lax.broadcasted_iota(jnp.int32, (T, T), 0)
                < lax.broadcasted_iota(jnp.int32, (T, T), 1))
        rank = jnp.dot(onehot.astype(jnp.bfloat16), triu.astype(jnp.bfloat16),
                       preferred_element_type=jnp.float32)
        myrank = jnp.sum(jnp.where(onehot, rank, 0.0), axis=0,
                         keepdims=True).astype(jnp.int32)
        slot_row = a_r * CE + myrank

        rows = lax.broadcasted_iota(jnp.int32, (N_EXP * CE, T), 0)
        P = (rows == slot_row).astype(jnp.bfloat16)
        p_ref[...] = P
        xb = x_ref[...].astype(jnp.bfloat16)

        drain = []
        for r in range(1, N_DEV):
            tgt = lax.rem(my + r, N_DEV)
            xpk[pl.ds(tgt * B, B), :] = jnp.dot(
                p_ref[pl.ds(tgt * B, B), :], xb,
                preferred_element_type=jnp.float32).astype(jnp.bfloat16)
            xs = pltpu.make_async_remote_copy(
                src_ref=xpk.at[pl.ds(tgt * B, B), :],
                dst_ref=xrx.at[pl.ds(my * B, B), :],
                send_sem=x_send.at[r - 1],
                recv_sem=x_recv.at[r - 1],
                device_id=(tgt,),
                device_id_type=pl.DeviceIdType.MESH,
            )
            xs.start()
            drain.append(xs)
        xpk[pl.ds(my * B, B), :] = jnp.dot(
            p_ref[pl.ds(my * B, B), :], xb,
            preferred_element_type=jnp.float32).astype(jnp.bfloat16)

        w1cp.wait()
        w2cp.wait()
        w1b[...] = w1f[...].astype(jnp.bfloat16)
        w2b[...] = w2f[...].astype(jnp.bfloat16)

        def compute_block(in_ref, out_ref_, base):
            for k in range(E_LOC):
                xa = in_ref[pl.ds(base + k * CE, CE), :]
                h = jnp.maximum(
                    jnp.dot(xa, w1b[k],
                            preferred_element_type=jnp.float32),
                    0.0).astype(jnp.bfloat16)
                y = jnp.dot(h, w2b[k], preferred_element_type=jnp.float32)
                out_ref_[pl.ds(base + k * CE, CE), :] = (
                    y.astype(jnp.bfloat16))

        compute_block(xpk, yrx, my * B)

        for r in range(1, N_DEV):
            src_pos = lax.rem(my - r + N_DEV, N_DEV)
            pltpu.make_async_remote_copy(
                src_ref=xpk.at[pl.ds(0, B), :],
                dst_ref=xrx.at[pl.ds(src_pos * B, B), :],
                send_sem=x_send.at[r - 1], recv_sem=x_recv.at[r - 1],
                device_id=(0,), device_id_type=pl.DeviceIdType.MESH,
            ).wait_recv()

            compute_block(xrx, yloc, src_pos * B)
            yd = pltpu.make_async_remote_copy(
                src_ref=yloc.at[pl.ds(src_pos * B, B), :],
                dst_ref=yrx.at[pl.ds(my * B, B), :],
                send_sem=y_send.at[r - 1],
                recv_sem=y_recv.at[(N_DEV - r) - 1],
                device_id=(src_pos,),
                device_id_type=pl.DeviceIdType.MESH,
            )
            yd.start()
            drain.append(yd)

        for rc in range(1, N_DEV):
            src_pos = lax.rem(my - rc + N_DEV, N_DEV)
            pltpu.make_async_remote_copy(
                src_ref=yloc.at[pl.ds(0, B), :],
                dst_ref=yrx.at[pl.ds(src_pos * B, B), :],
                send_sem=y_send.at[rc - 1],
                recv_sem=y_recv.at[rc - 1],
                device_id=(0,),
                device_id_type=pl.DeviceIdType.MESH,
            ).wait_recv()

        out_ref[...] = lax.dot_general(
            P, yrx[...], dimension_numbers=(((0,), (0,)), ((), ())),
            preferred_element_type=jnp.float32)

        for d in drain:
            d.wait_send()

    return pl.pallas_call(
        body,
        out_shape=jax.ShapeDtypeStruct((T, D), jnp.float32),
        in_specs=[pl.BlockSpec(memory_space=pltpu.VMEM),
                  pl.BlockSpec(memory_space=pltpu.VMEM),
                  pl.BlockSpec(memory_space=pltpu.MemorySpace.HBM),
                  pl.BlockSpec(memory_space=pltpu.MemorySpace.HBM)],
        out_specs=pl.BlockSpec(memory_space=pltpu.VMEM),
        scratch_shapes=[
            pltpu.VMEM((N_EXP * CE, D), jnp.bfloat16),
            pltpu.VMEM((N_EXP * CE, D), jnp.bfloat16),
            pltpu.VMEM((N_EXP * CE, D), jnp.bfloat16),
            pltpu.VMEM((N_EXP * CE, D), jnp.bfloat16),
            pltpu.VMEM((N_EXP * CE, T), jnp.bfloat16),
            pltpu.VMEM((E_LOC, D, F), jnp.float32),
            pltpu.VMEM((E_LOC, F, D), jnp.float32),
            pltpu.VMEM((E_LOC, D, F), jnp.bfloat16),
            pltpu.VMEM((E_LOC, F, D), jnp.bfloat16),
            pltpu.SemaphoreType.DMA((N_DEV - 1,)),
            pltpu.SemaphoreType.DMA((N_DEV - 1,)),
            pltpu.SemaphoreType.DMA((N_DEV - 1,)),
            pltpu.SemaphoreType.DMA((N_DEV - 1,)),
            pltpu.SemaphoreType.DMA,
            pltpu.SemaphoreType.DMA,
        ],
        compiler_params=pltpu.CompilerParams(collective_id=0),
    )(x, a_row, W1, W2)


# device time: 22177 ns/iter; 1.1848x vs baseline; 1.0926x over previous
import jax
import jax.numpy as jnp
from jax import lax
from jax.experimental import pallas as pl
from jax.experimental.pallas import tpu as pltpu

N_DEV = 4
T = 512
D = 512
F = 1024
E_LOC = 2
N_EXP = N_DEV * E_LOC
CE = 80
B = E_LOC * CE


def kernel(x, assign, W1, W2):
    a_row = assign.reshape(1, T)
    xb16 = x.astype(jnp.bfloat16)
    W1b16 = W1.astype(jnp.bfloat16)
    W2b16 = W2.astype(jnp.bfloat16)

    def body(x_ref, a_ref, w1_ref, w2_ref, out_ref,
             xpk, xrx, yloc, yrx, p_ref, w1b, w2b,
             x_send, x_recv, y_send, y_recv, w1_sem, w2_sem):
        my = lax.axis_index("i")

        w1cp = pltpu.make_async_copy(w1_ref, w1b, w1_sem)
        w2cp = pltpu.make_async_copy(w2_ref, w2b, w2_sem)
        w1cp.start()
        w2cp.start()

        barrier = pltpu.get_barrier_semaphore()
        for r in range(1, N_DEV):
            tgt = lax.rem(my + r, N_DEV)
            pl.semaphore_signal(barrier, inc=1, device_id=(tgt,),
                                device_id_type=pl.DeviceIdType.MESH)
        pl.semaphore_wait(barrier, N_DEV - 1)

        a_r = a_ref[...]
        onehot = (a_r == lax.broadcasted_iota(jnp.int32, (N_EXP, T), 0))
        triu = (lax.broadcasted_iota(jnp.int32, (T, T), 0)
                < lax.broadcasted_iota(jnp.int32, (T, T), 1))
        rank = jnp.dot(onehot.astype(jnp.bfloat16), triu.astype(jnp.bfloat16),
                       preferred_element_type=jnp.float32)
        myrank = jnp.sum(jnp.where(onehot, rank, 0.0), axis=0,
                         keepdims=True).astype(jnp.int32)
        slot_row = a_r * CE + myrank

        rows = lax.broadcasted_iota(jnp.int32, (N_EXP * CE, T), 0)
        P = (rows == slot_row).astype(jnp.bfloat16)
        p_ref[...] = P
        xb = x_ref[...]

        drain = []
        for r in range(1, N_DEV):
            tgt = lax.rem(my + r, N_DEV)
            xpk[pl.ds(tgt * B, B), :] = jnp.dot(
                p_ref[pl.ds(tgt * B, B), :], xb,
                preferred_element_type=jnp.float32).astype(jnp.bfloat16)
            xs = pltpu.make_async_remote_copy(
                src_ref=xpk.at[pl.ds(tgt * B, B), :],
                dst_ref=xrx.at[pl.ds(my * B, B), :],
                send_sem=x_send.at[r - 1],
                recv_sem=x_recv.at[r - 1],
                device_id=(tgt,),
                device_id_type=pl.DeviceIdType.MESH,
            )
            xs.start()
            drain.append(xs)
        xpk[pl.ds(my * B, B), :] = jnp.dot(
            p_ref[pl.ds(my * B, B), :], xb,
            preferred_element_type=jnp.float32).astype(jnp.bfloat16)

        w1cp.wait()
        w2cp.wait()

        def compute_block(in_ref, out_ref_, base):
            for k in range(E_LOC):
                xa = in_ref[pl.ds(base + k * CE, CE), :]
                h = jnp.maximum(
                    jnp.dot(xa, w1b[k],
                            preferred_element_type=jnp.float32),
                    0.0).astype(jnp.bfloat16)
                y = jnp.dot(h, w2b[k], preferred_element_type=jnp.float32)
                out_ref_[pl.ds(base + k * CE, CE), :] = (
                    y.astype(jnp.bfloat16))

        compute_block(xpk, yrx, my * B)

        for r in range(1, N_DEV):
            src_pos = lax.rem(my - r + N_DEV, N_DEV)
            pltpu.make_async_remote_copy(
                src_ref=xpk.at[pl.ds(0, B), :],
                dst_ref=xrx.at[pl.ds(src_pos * B, B), :],
                send_sem=x_send.at[r - 1], recv_sem=x_recv.at[r - 1],
                device_id=(0,), device_id_type=pl.DeviceIdType.MESH,
            ).wait_recv()

            compute_block(xrx, yloc, src_pos * B)
            yd = pltpu.make_async_remote_copy(
                src_ref=yloc.at[pl.ds(src_pos * B, B), :],
                dst_ref=yrx.at[pl.ds(my * B, B), :],
                send_sem=y_send.at[r - 1],
                recv_sem=y_recv.at[(N_DEV - r) - 1],
                device_id=(src_pos,),
                device_id_type=pl.DeviceIdType.MESH,
            )
            yd.start()
            drain.append(yd)

        for rc in range(1, N_DEV):
            src_pos = lax.rem(my - rc + N_DEV, N_DEV)
            pltpu.make_async_remote_copy(
                src_ref=yloc.at[pl.ds(0, B), :],
                dst_ref=yrx.at[pl.ds(src_pos * B, B), :],
                send_sem=y_send.at[rc - 1],
                recv_sem=y_recv.at[rc - 1],
                device_id=(0,),
                device_id_type=pl.DeviceIdType.MESH,
            ).wait_recv()

        out_ref[...] = lax.dot_general(
            P, yrx[...], dimension_numbers=(((0,), (0,)), ((), ())),
            preferred_element_type=jnp.float32).astype(jnp.bfloat16)

        for d in drain:
            d.wait_send()

    return pl.pallas_call(
        body,
        out_shape=jax.ShapeDtypeStruct((T, D), jnp.bfloat16),
        in_specs=[pl.BlockSpec(memory_space=pltpu.VMEM),
                  pl.BlockSpec(memory_space=pltpu.VMEM),
                  pl.BlockSpec(memory_space=pltpu.MemorySpace.HBM),
                  pl.BlockSpec(memory_space=pltpu.MemorySpace.HBM)],
        out_specs=pl.BlockSpec(memory_space=pltpu.VMEM),
        scratch_shapes=[
            pltpu.VMEM((N_EXP * CE, D), jnp.bfloat16),
            pltpu.VMEM((N_EXP * CE, D), jnp.bfloat16),
            pltpu.VMEM((N_EXP * CE, D), jnp.bfloat16),
            pltpu.VMEM((N_EXP * CE, D), jnp.bfloat16),
            pltpu.VMEM((N_EXP * CE, T), jnp.bfloat16),
            pltpu.VMEM((E_LOC, D, F), jnp.bfloat16),
            pltpu.VMEM((E_LOC, F, D), jnp.bfloat16),
            pltpu.SemaphoreType.DMA((N_DEV - 1,)),
            pltpu.SemaphoreType.DMA((N_DEV - 1,)),
            pltpu.SemaphoreType.DMA((N_DEV - 1,)),
            pltpu.SemaphoreType.DMA((N_DEV - 1,)),
            pltpu.SemaphoreType.DMA,
            pltpu.SemaphoreType.DMA,
        ],
        compiler_params=pltpu.CompilerParams(collective_id=0),
    )(xb16, a_row, W1b16, W2b16)


# device time: 21046 ns/iter; 1.2485x vs baseline; 1.0537x over previous
import jax
import jax.numpy as jnp
from jax import lax
from jax.experimental import pallas as pl
from jax.experimental.pallas import tpu as pltpu

N_DEV = 4
T = 512
D = 512
F = 1024
E_LOC = 2
N_EXP = N_DEV * E_LOC
CE = 80
B = E_LOC * CE


def kernel(x, assign, W1, W2):
    a_row = assign.reshape(1, T)
    xb16 = x.astype(jnp.bfloat16)
    W1b16 = W1.astype(jnp.bfloat16)
    W2b16 = W2.astype(jnp.bfloat16)

    def body(x_ref, a_ref, w1_ref, w2_ref, out_ref,
             xpk, xrx, yloc, yrx, p_ref, w1b, w2b,
             x_send, x_recv, y_send, y_recv, w1_sem, w2_sem):
        my = lax.axis_index("i")

        w1cp = pltpu.make_async_copy(w1_ref, w1b, w1_sem)
        w2cp = pltpu.make_async_copy(w2_ref, w2b, w2_sem)
        w1cp.start()
        w2cp.start()

        barrier = pltpu.get_barrier_semaphore()
        for r in range(1, N_DEV):
            tgt = lax.rem(my + r, N_DEV)
            pl.semaphore_signal(barrier, inc=1, device_id=(tgt,),
                                device_id_type=pl.DeviceIdType.MESH)
        pl.semaphore_wait(barrier, N_DEV - 1)

        a_r = a_ref[...]
        onehot = (a_r == lax.broadcasted_iota(jnp.int32, (N_EXP, T), 0))
        triu = (lax.broadcasted_iota(jnp.int32, (T, T), 0)
                < lax.broadcasted_iota(jnp.int32, (T, T), 1))
        rank = jnp.dot(onehot.astype(jnp.bfloat16), triu.astype(jnp.bfloat16),
                       preferred_element_type=jnp.float32)
        myrank = jnp.sum(jnp.where(onehot, rank, 0.0), axis=0,
                         keepdims=True).astype(jnp.int32)
        slot_row = a_r * CE + myrank

        rows = lax.broadcasted_iota(jnp.int32, (N_EXP * CE, T), 0)
        P = (rows == slot_row).astype(jnp.bfloat16)
        p_ref[...] = P
        xb = x_ref[...]

        drain = []
        for r in range(1, N_DEV):
            tgt = lax.rem(my + r, N_DEV)
            xpk[pl.ds(tgt * B, B), :] = jnp.dot(
                p_ref[pl.ds(tgt * B, B), :], xb,
                preferred_element_type=jnp.float32).astype(jnp.bfloat16)
            for k in range(E_LOC):
                xs = pltpu.make_async_remote_copy(
                    src_ref=xpk.at[pl.ds(tgt * B + k * CE, CE), :],
                    dst_ref=xrx.at[pl.ds(my * B + k * CE, CE), :],
                    send_sem=x_send.at[(r - 1) * E_LOC + k],
                    recv_sem=x_recv.at[(r - 1) * E_LOC + k],
                    device_id=(tgt,),
                    device_id_type=pl.DeviceIdType.MESH,
                )
                xs.start()
                drain.append(xs)
        xpk[pl.ds(my * B, B), :] = jnp.dot(
            p_ref[pl.ds(my * B, B), :], xb,
            preferred_element_type=jnp.float32).astype(jnp.bfloat16)

        w1cp.wait()
        w2cp.wait()

        def compute_sub(in_ref, out_ref_, base, k):
            xa = in_ref[pl.ds(base + k * CE, CE), :]
            h = jnp.maximum(
                jnp.dot(xa, w1b[k], preferred_element_type=jnp.float32),
                0.0).astype(jnp.bfloat16)
            y = jnp.dot(h, w2b[k], preferred_element_type=jnp.float32)
            out_ref_[pl.ds(base + k * CE, CE), :] = y.astype(jnp.bfloat16)

        for k in range(E_LOC):
            compute_sub(xpk, yrx, my * B, k)

        for r in range(1, N_DEV):
            src_pos = lax.rem(my - r + N_DEV, N_DEV)
            for k in range(E_LOC):
                pltpu.make_async_remote_copy(
                    src_ref=xpk.at[pl.ds(0, CE), :],
                    dst_ref=xrx.at[pl.ds(src_pos * B + k * CE, CE), :],
                    send_sem=x_send.at[(r - 1) * E_LOC + k],
                    recv_sem=x_recv.at[(r - 1) * E_LOC + k],
                    device_id=(0,), device_id_type=pl.DeviceIdType.MESH,
                ).wait_recv()

                compute_sub(xrx, yloc, src_pos * B, k)
                yd = pltpu.make_async_remote_copy(
                    src_ref=yloc.at[pl.ds(src_pos * B + k * CE, CE), :],
                    dst_ref=yrx.at[pl.ds(my * B + k * CE, CE), :],
                    send_sem=y_send.at[(r - 1) * E_LOC + k],
                    recv_sem=y_recv.at[(N_DEV - r - 1) * E_LOC + k],
                    device_id=(src_pos,),
                    device_id_type=pl.DeviceIdType.MESH,
                )
                yd.start()
                drain.append(yd)

        for rc in range(1, N_DEV):
            src_pos = lax.rem(my - rc + N_DEV, N_DEV)
            for k in range(E_LOC):
                pltpu.make_async_remote_copy(
                    src_ref=yloc.at[pl.ds(0, CE), :],
                    dst_ref=yrx.at[pl.ds(src_pos * B + k * CE, CE), :],
                    send_sem=y_send.at[(rc - 1) * E_LOC + k],
                    recv_sem=y_recv.at[(rc - 1) * E_LOC + k],
                    device_id=(0,),
                    device_id_type=pl.DeviceIdType.MESH,
                ).wait_recv()

        out_ref[...] = lax.dot_general(
            P, yrx[...], dimension_numbers=(((0,), (0,)), ((), ())),
            preferred_element_type=jnp.float32).astype(jnp.bfloat16)

        for d in drain:
            d.wait_send()

    return pl.pallas_call(
        body,
        out_shape=jax.ShapeDtypeStruct((T, D), jnp.bfloat16),
        in_specs=[pl.BlockSpec(memory_space=pltpu.VMEM),
                  pl.BlockSpec(memory_space=pltpu.VMEM),
                  pl.BlockSpec(memory_space=pltpu.MemorySpace.HBM),
                  pl.BlockSpec(memory_space=pltpu.MemorySpace.HBM)],
        out_specs=pl.BlockSpec(memory_space=pltpu.VMEM),
        scratch_shapes=[
            pltpu.VMEM((N_EXP * CE, D), jnp.bfloat16),
            pltpu.VMEM((N_EXP * CE, D), jnp.bfloat16),
            pltpu.VMEM((N_EXP * CE, D), jnp.bfloat16),
            pltpu.VMEM((N_EXP * CE, D), jnp.bfloat16),
            pltpu.VMEM((N_EXP * CE, T), jnp.bfloat16),
            pltpu.VMEM((E_LOC, D, F), jnp.bfloat16),
            pltpu.VMEM((E_LOC, F, D), jnp.bfloat16),
            pltpu.SemaphoreType.DMA(((N_DEV - 1) * E_LOC,)),
            pltpu.SemaphoreType.DMA(((N_DEV - 1) * E_LOC,)),
            pltpu.SemaphoreType.DMA(((N_DEV - 1) * E_LOC,)),
            pltpu.SemaphoreType.DMA(((N_DEV - 1) * E_LOC,)),
            pltpu.SemaphoreType.DMA,
            pltpu.SemaphoreType.DMA,
        ],
        compiler_params=pltpu.CompilerParams(collective_id=0),
    )(xb16, a_row, W1b16, W2b16)


# device time: 20680 ns/iter; 1.2706x vs baseline; 1.0177x over previous
import jax
import jax.numpy as jnp
from jax import lax
from jax.experimental import pallas as pl
from jax.experimental.pallas import tpu as pltpu

N_DEV = 4
T = 512
D = 512
F = 1024
E_LOC = 2
N_EXP = N_DEV * E_LOC
CE = 80
B = E_LOC * CE


def kernel(x, assign, W1, W2):
    a_row = assign.reshape(1, T)
    xb16 = x.astype(jnp.bfloat16)
    W1b16 = W1.astype(jnp.bfloat16)
    W2b16 = W2.astype(jnp.bfloat16)

    def body(x_ref, a_ref, w1_ref, w2_ref, out_ref,
             xpk, xrx, yloc, yrx, p_ref, w1b, w2b,
             x_send, x_recv, y_send, y_recv, w1_sem, w2_sem):
        my = lax.axis_index("i")

        w1cp = pltpu.make_async_copy(w1_ref, w1b, w1_sem)
        w2cp = pltpu.make_async_copy(w2_ref, w2b, w2_sem)
        w1cp.start()
        w2cp.start()

        barrier = pltpu.get_barrier_semaphore()
        for r in range(1, N_DEV):
            tgt = lax.rem(my + r, N_DEV)
            pl.semaphore_signal(barrier, inc=1, device_id=(tgt,),
                                device_id_type=pl.DeviceIdType.MESH)
        pl.semaphore_wait(barrier, N_DEV - 1)

        a_r = a_ref[...]
        onehot = (a_r == lax.broadcasted_iota(jnp.int32, (N_EXP, T), 0))
        triu = (lax.broadcasted_iota(jnp.int32, (T, T), 0)
                < lax.broadcasted_iota(jnp.int32, (T, T), 1))
        rank = jnp.dot(onehot.astype(jnp.bfloat16), triu.astype(jnp.bfloat16),
                       preferred_element_type=jnp.float32)
        myrank = jnp.sum(jnp.where(onehot, rank, 0.0), axis=0,
                         keepdims=True).astype(jnp.int32)
        slot_row = a_r * CE + myrank

        rows = lax.broadcasted_iota(jnp.int32, (N_EXP * CE, T), 0)
        P = (rows == slot_row).astype(jnp.bfloat16)
        p_ref[...] = P
        xb = x_ref[...]

        drain = []
        for r in range(1, N_DEV):
            tgt = lax.rem(my + r, N_DEV)
            xpk[pl.ds(tgt * B, B), :] = jnp.dot(
                p_ref[pl.ds(tgt * B, B), :], xb,
                preferred_element_type=jnp.float32).astype(jnp.bfloat16)
            for k in range(E_LOC):
                xs = pltpu.make_async_remote_copy(
                    src_ref=xpk.at[pl.ds(tgt * B + k * CE, CE), :],
                    dst_ref=xrx.at[pl.ds(my * B + k * CE, CE), :],
                    send_sem=x_send.at[(r - 1) * E_LOC + k],
                    recv_sem=x_recv.at[(r - 1) * E_LOC + k],
                    device_id=(tgt,),
                    device_id_type=pl.DeviceIdType.MESH,
                )
                xs.start()
                drain.append(xs)
        xpk[pl.ds(my * B, B), :] = jnp.dot(
            p_ref[pl.ds(my * B, B), :], xb,
            preferred_element_type=jnp.float32).astype(jnp.bfloat16)

        w1cp.wait()
        w2cp.wait()

        def compute_sub(in_ref, out_ref_, base, k):
            xa = in_ref[pl.ds(base + k * CE, CE), :]
            h = jnp.maximum(
                jnp.dot(xa, w1b[k], preferred_element_type=jnp.float32),
                0.0).astype(jnp.bfloat16)
            y = jnp.dot(h, w2b[k], preferred_element_type=jnp.float32)
            out_ref_[pl.ds(base + k * CE, CE), :] = y.astype(jnp.bfloat16)

        for k in range(E_LOC):
            compute_sub(xpk, yrx, my * B, k)
        acc_out = lax.dot_general(
            p_ref[pl.ds(my * B, B), :], yrx[pl.ds(my * B, B), :],
            dimension_numbers=(((0,), (0,)), ((), ())),
            preferred_element_type=jnp.float32)

        for r in range(1, N_DEV):
            src_pos = lax.rem(my - r + N_DEV, N_DEV)
            for k in range(E_LOC):
                pltpu.make_async_remote_copy(
                    src_ref=xpk.at[pl.ds(0, CE), :],
                    dst_ref=xrx.at[pl.ds(src_pos * B + k * CE, CE), :],
                    send_sem=x_send.at[(r - 1) * E_LOC + k],
                    recv_sem=x_recv.at[(r - 1) * E_LOC + k],
                    device_id=(0,), device_id_type=pl.DeviceIdType.MESH,
                ).wait_recv()

                compute_sub(xrx, yloc, src_pos * B, k)
                yd = pltpu.make_async_remote_copy(
                    src_ref=yloc.at[pl.ds(src_pos * B + k * CE, CE), :],
                    dst_ref=yrx.at[pl.ds(my * B + k * CE, CE), :],
                    send_sem=y_send.at[(r - 1) * E_LOC + k],
                    recv_sem=y_recv.at[(N_DEV - r - 1) * E_LOC + k],
                    device_id=(src_pos,),
                    device_id_type=pl.DeviceIdType.MESH,
                )
                yd.start()
                drain.append(yd)

        for rc in range(1, N_DEV):
            src_pos = lax.rem(my - rc + N_DEV, N_DEV)
            for k in range(E_LOC):
                pltpu.make_async_remote_copy(
                    src_ref=yloc.at[pl.ds(0, CE), :],
                    dst_ref=yrx.at[pl.ds(src_pos * B + k * CE, CE), :],
                    send_sem=y_send.at[(rc - 1) * E_LOC + k],
                    recv_sem=y_recv.at[(rc - 1) * E_LOC + k],
                    device_id=(0,),
                    device_id_type=pl.DeviceIdType.MESH,
                ).wait_recv()
            acc_out = acc_out + lax.dot_general(
                p_ref[pl.ds(src_pos * B, B), :], yrx[pl.ds(src_pos * B, B), :],
                dimension_numbers=(((0,), (0,)), ((), ())),
                preferred_element_type=jnp.float32)

        out_ref[...] = acc_out.astype(jnp.bfloat16)

        for d in drain:
            d.wait_send()

    return pl.pallas_call(
        body,
        out_shape=jax.ShapeDtypeStruct((T, D), jnp.bfloat16),
        in_specs=[pl.BlockSpec(memory_space=pltpu.VMEM),
                  pl.BlockSpec(memory_space=pltpu.VMEM),
                  pl.BlockSpec(memory_space=pltpu.MemorySpace.HBM),
                  pl.BlockSpec(memory_space=pltpu.MemorySpace.HBM)],
        out_specs=pl.BlockSpec(memory_space=pltpu.VMEM),
        scratch_shapes=[
            pltpu.VMEM((N_EXP * CE, D), jnp.bfloat16),
            pltpu.VMEM((N_EXP * CE, D), jnp.bfloat16),
            pltpu.VMEM((N_EXP * CE, D), jnp.bfloat16),
            pltpu.VMEM((N_EXP * CE, D), jnp.bfloat16),
            pltpu.VMEM((N_EXP * CE, T), jnp.bfloat16),
            pltpu.VMEM((E_LOC, D, F), jnp.bfloat16),
            pltpu.VMEM((E_LOC, F, D), jnp.bfloat16),
            pltpu.SemaphoreType.DMA(((N_DEV - 1) * E_LOC,)),
            pltpu.SemaphoreType.DMA(((N_DEV - 1) * E_LOC,)),
            pltpu.SemaphoreType.DMA(((N_DEV - 1) * E_LOC,)),
            pltpu.SemaphoreType.DMA(((N_DEV - 1) * E_LOC,)),
            pltpu.SemaphoreType.DMA,
            pltpu.SemaphoreType.DMA,
        ],
        compiler_params=pltpu.CompilerParams(collective_id=0),
    )(xb16, a_row, W1b16, W2b16)
